# Initial kernel scaffold; baseline (speedup 1.0000x reference)
import jax
import jax.numpy as jnp
from jax.experimental import pallas as pl


def kernel(z, codebooks):
    raise NotImplementedError("write your pallas kernel here")


# speedup vs baseline: 3.9664x; 3.9664x over previous
import functools

import jax
import jax.numpy as jnp
from jax.experimental import pallas as pl
from jax.experimental.pallas import tpu as pltpu

_NUM_LEVELS = 4
_K = 512
_D = 32
_B = 4096
_BB = 512


def _rq_body(z_ref, cb_ref, cbt_ref, quant_ref, lev_ref, loss_ref):
    r = z_ref[...]
    acc = jnp.zeros_like(r)
    partial = jnp.float32(0.0)
    for l in range(_NUM_LEVELS):
        cb = cb_ref[l]
        cbt = cbt_ref[l]
        cnorm = jnp.sum(cbt * cbt, axis=0)
        rc = jax.lax.dot_general(
            r, cbt, (((1,), (0,)), ((), ())),
            precision=jax.lax.Precision.HIGHEST,
            preferred_element_type=jnp.float32)
        scores = cnorm[None, :] - 2.0 * rc
        m = jnp.min(scores, axis=1, keepdims=True)
        iota = jax.lax.broadcasted_iota(jnp.int32, scores.shape, 1)
        idx = jnp.min(jnp.where(scores == m, iota, _K), axis=1,
                      keepdims=True)
        onehot = (iota == idx).astype(jnp.float32)
        chosen = jax.lax.dot_general(
            onehot, cb, (((1,), (0,)), ((), ())),
            precision=jax.lax.Precision.HIGHEST,
            preferred_element_type=jnp.float32)
        lev_ref[l] = chosen
        r = r - chosen
        acc = acc + chosen
        partial = partial + jnp.sum(r * r)
    quant_ref[...] = acc

    @pl.when(pl.program_id(0) == 0)
    def _init():
        loss_ref[0] = 0.0

    loss_ref[0] += partial


@jax.jit
def kernel(z, codebooks):
    grid = _B // _BB
    quant, levels, loss = pl.pallas_call(
        _rq_body,
        grid=(grid,),
        in_specs=[
            pl.BlockSpec((_BB, _D), lambda i: (i, 0)),
            pl.BlockSpec((_NUM_LEVELS, _K, _D), lambda i: (0, 0, 0)),
            pl.BlockSpec((_NUM_LEVELS, _D, _K), lambda i: (0, 0, 0)),
        ],
        out_specs=[
            pl.BlockSpec((_BB, _D), lambda i: (i, 0)),
            pl.BlockSpec((_NUM_LEVELS, _BB, _D), lambda i: (0, i, 0)),
            pl.BlockSpec(memory_space=pltpu.SMEM),
        ],
        out_shape=[
            jax.ShapeDtypeStruct((_B, _D), jnp.float32),
            jax.ShapeDtypeStruct((_NUM_LEVELS, _B, _D), jnp.float32),
            jax.ShapeDtypeStruct((1,), jnp.float32),
        ],
    )(z, codebooks, jnp.transpose(codebooks, (0, 2, 1)))
    rq_loss = (loss[0] / jnp.float32(_B * _D))
    return (quant, levels, rq_loss)


# speedup vs baseline: 6.6854x; 1.6855x over previous
import functools

import jax
import jax.numpy as jnp
from jax.experimental import pallas as pl
from jax.experimental.pallas import tpu as pltpu

_NUM_LEVELS = 4
_K = 512
_D = 32
_B = 4096
_BB = 1024


def _rq_body(z_ref, cb_ref, cbt_ref, quant_ref, lev_ref, loss_ref):
    r = z_ref[...]
    acc = jnp.zeros_like(r)
    partial = jnp.float32(0.0)
    for l in range(_NUM_LEVELS):
        cb = cb_ref[l]
        cbt = cbt_ref[l]
        cnorm = jnp.sum(cbt * cbt, axis=0)
        rc = jax.lax.dot_general(
            r, cbt, (((1,), (0,)), ((), ())),
            precision=jax.lax.Precision.HIGHEST,
            preferred_element_type=jnp.float32)
        scores = cnorm[None, :] - 2.0 * rc
        m = jnp.min(scores, axis=1, keepdims=True)
        iota = jax.lax.broadcasted_iota(jnp.int32, scores.shape, 1)
        idx = jnp.min(jnp.where(scores == m, iota, _K), axis=1,
                      keepdims=True)
        onehot = (iota == idx).astype(jnp.bfloat16)
        cb_hi = cb.astype(jnp.bfloat16)
        rem = cb - cb_hi.astype(jnp.float32)
        cb_mid = rem.astype(jnp.bfloat16)
        cb_lo = (rem - cb_mid.astype(jnp.float32)).astype(jnp.bfloat16)
        dims = (((1,), (0,)), ((), ()))
        chosen = jax.lax.dot_general(
            onehot, cb_hi, dims, preferred_element_type=jnp.float32)
        chosen = chosen + jax.lax.dot_general(
            onehot, cb_mid, dims, preferred_element_type=jnp.float32)
        chosen = chosen + jax.lax.dot_general(
            onehot, cb_lo, dims, preferred_element_type=jnp.float32)
        lev_ref[l] = chosen
        r = r - chosen
        acc = acc + chosen
        partial = partial + jnp.sum(r * r)
    quant_ref[...] = acc

    @pl.when(pl.program_id(0) == 0)
    def _init():
        loss_ref[0] = 0.0

    loss_ref[0] += partial


@jax.jit
def kernel(z, codebooks):
    grid = _B // _BB
    quant, levels, loss = pl.pallas_call(
        _rq_body,
        grid=(grid,),
        in_specs=[
            pl.BlockSpec((_BB, _D), lambda i: (i, 0)),
            pl.BlockSpec((_NUM_LEVELS, _K, _D), lambda i: (0, 0, 0)),
            pl.BlockSpec((_NUM_LEVELS, _D, _K), lambda i: (0, 0, 0)),
        ],
        out_specs=[
            pl.BlockSpec((_BB, _D), lambda i: (i, 0)),
            pl.BlockSpec((_NUM_LEVELS, _BB, _D), lambda i: (0, i, 0)),
            pl.BlockSpec(memory_space=pltpu.SMEM),
        ],
        out_shape=[
            jax.ShapeDtypeStruct((_B, _D), jnp.float32),
            jax.ShapeDtypeStruct((_NUM_LEVELS, _B, _D), jnp.float32),
            jax.ShapeDtypeStruct((1,), jnp.float32),
        ],
    )(z, codebooks, jnp.transpose(codebooks, (0, 2, 1)))
    rq_loss = (loss[0] / jnp.float32(_B * _D))
    return (quant, levels, rq_loss)


# speedup vs baseline: 10.0962x; 1.5102x over previous
import functools

import jax
import jax.numpy as jnp
from jax.experimental import pallas as pl
from jax.experimental.pallas import tpu as pltpu

_NUM_LEVELS = 4
_K = 512
_D = 32
_B = 4096
_BB = 4096


def _rq_body(z_ref, cb3_ref, ct6_ref, cnorm_ref, quant_ref, lev_ref, loss_ref):
    r = z_ref[...]
    acc = jnp.zeros_like(r)
    partial = jnp.float32(0.0)
    for l in range(_NUM_LEVELS):
        cb3 = cb3_ref[l]
        ct6 = ct6_ref[l]
        cnorm = cnorm_ref[l]
        r_hi = r.astype(jnp.bfloat16)
        rrem = r - r_hi.astype(jnp.float32)
        r_mid = rrem.astype(jnp.bfloat16)
        r_lo = (rrem - r_mid.astype(jnp.float32)).astype(jnp.bfloat16)
        r6 = jnp.concatenate([r_hi, r_hi, r_mid, r_hi, r_lo, r_mid],
                             axis=1)
        rc2 = jax.lax.dot_general(
            r6, ct6, (((1,), (0,)), ((), ())),
            preferred_element_type=jnp.float32)
        scores = cnorm + rc2
        m = jnp.min(scores, axis=1, keepdims=True)
        iota = jax.lax.broadcasted_iota(jnp.int32, scores.shape, 1)
        idx = jnp.min(jnp.where(scores == m, iota, _K), axis=1,
                      keepdims=True)
        onehot = (iota == idx).astype(jnp.bfloat16)
        g = jax.lax.dot_general(
            onehot, cb3, (((1,), (0,)), ((), ())),
            preferred_element_type=jnp.float32)
        chosen = (g[:, :_D] + g[:, _D:2 * _D]) + g[:, 2 * _D:]
        lev_ref[l] = chosen
        r = r - chosen
        acc = acc + chosen
        partial = partial + jnp.sum(r * r)
    quant_ref[...] = acc

    @pl.when(pl.program_id(0) == 0)
    def _init():
        loss_ref[0] = 0.0

    loss_ref[0] += partial


@jax.jit
def kernel(z, codebooks):
    grid = _B // _BB
    _call = pl.pallas_call(
        _rq_body,
        grid=(grid,),
        in_specs=[
            pl.BlockSpec((_BB, _D), lambda i: (i, 0)),
            pl.BlockSpec((_NUM_LEVELS, _K, 3 * _D), lambda i: (0, 0, 0)),
            pl.BlockSpec((_NUM_LEVELS, 6 * _D, _K), lambda i: (0, 0, 0)),
            pl.BlockSpec((_NUM_LEVELS, 1, _K), lambda i: (0, 0, 0)),
        ],
        out_specs=[
            pl.BlockSpec((_BB, _D), lambda i: (i, 0)),
            pl.BlockSpec((_NUM_LEVELS, _BB, _D), lambda i: (0, i, 0)),
            pl.BlockSpec(memory_space=pltpu.SMEM),
        ],
        out_shape=[
            jax.ShapeDtypeStruct((_B, _D), jnp.float32),
            jax.ShapeDtypeStruct((_NUM_LEVELS, _B, _D), jnp.float32),
            jax.ShapeDtypeStruct((1,), jnp.float32),
        ],
    )
    cb_hi = codebooks.astype(jnp.bfloat16)
    rem = codebooks - cb_hi.astype(jnp.float32)
    cb_mid = rem.astype(jnp.bfloat16)
    cb_lo = (rem - cb_mid.astype(jnp.float32)).astype(jnp.bfloat16)
    cb3 = jnp.concatenate([cb_hi, cb_mid, cb_lo], axis=2)
    cbt2 = jnp.transpose(codebooks, (0, 2, 1)) * -2.0
    ct_hi = cbt2.astype(jnp.bfloat16)
    trem = cbt2 - ct_hi.astype(jnp.float32)
    ct_mid = trem.astype(jnp.bfloat16)
    ct_lo = (trem - ct_mid.astype(jnp.float32)).astype(jnp.bfloat16)
    ct6 = jnp.concatenate([ct_hi, ct_mid, ct_hi, ct_lo, ct_hi, ct_mid],
                          axis=1)
    cnorm = jnp.sum(jnp.transpose(codebooks, (0, 2, 1)) ** 2,
                    axis=1, keepdims=True)
    quant, levels, loss = _call(z, cb3, ct6, cnorm)
    rq_loss = (loss[0] / jnp.float32(_B * _D))
    return (quant, levels, rq_loss)
